# Initial kernel scaffold; baseline (speedup 1.0000x reference)
import jax
import jax.numpy as jnp
from jax.experimental import pallas as pl


def kernel(x, edge_index, edge_weight):
    raise NotImplementedError("write your pallas kernel here")


# speedup vs baseline: 16.3927x; 16.3927x over previous
import functools

import jax
import jax.numpy as jnp
from jax import lax
from jax.experimental import pallas as pl
from jax.experimental.pallas import tpu as pltpu
from jax.experimental.pallas import tpu_sc as plsc

N = 10000
NPAD = 10240
E = 320000
D = 128
NC = 2
NS = 16
NW = NC * NS
EPW = E // NW
CHUNK = 80
NCHUNK = EPW // CHUNK
ROWS_PER_TILE = NPAD // NS
WB = CHUNK
DEGW = 10


def _sc_body(x_hbm, src_hbm, dst_hbm, w_hbm, p_hbm, deg_hbm,
             out_acc, deg_acc, src_v, dst_v, w_v, rows_v, idx_v):
    c = lax.axis_index("c")
    s = lax.axis_index("s")
    wid = s * NC + c
    base = wid * EPW

    zeros16 = jnp.zeros((16,), jnp.float32)

    def _zero_rows(r, _):
        for j in range(D // 16):
            rows_v[r, pl.ds(j * 16, 16)] = zeros16
        return _
    lax.fori_loop(0, WB, _zero_rows, 0)
    for g in range(1024 // 16):
        w_v[pl.ds(g * 16, 16)] = zeros16

    for k in range(ROWS_PER_TILE // WB):
        r0 = s * ROWS_PER_TILE + k * WB
        pltpu.sync_copy(rows_v, out_acc.at[pl.ds(r0, WB)])

    @pl.when(s < DEGW)
    def _():
        pltpu.sync_copy(w_v.at[pl.ds(0, 1000)],
                        deg_acc.at[pl.ds(s * 1000, 1000)])

    plsc.subcore_barrier()

    pltpu.sync_copy(src_hbm.at[pl.ds(base, EPW)], src_v)
    pltpu.sync_copy(dst_hbm.at[pl.ds(base, EPW)], dst_v)
    pltpu.sync_copy(w_hbm.at[pl.ds(base, EPW)], w_v)

    def _deg_chunk(ci, _):
        off = ci * CHUNK
        for j in range(CHUNK // 16):
            idx_v[pl.ds(j * 16, 16)] = src_v[pl.ds(off + j * 16, 16)]
        pltpu.sync_copy(w_v.at[pl.ds(off, CHUNK)],
                        deg_acc.at[idx_v], add=True)
        return _
    lax.fori_loop(0, NCHUNK, _deg_chunk, 0)

    def _row_chunk(ci, _):
        off = ci * CHUNK
        for j in range(CHUNK // 16):
            idx_v[pl.ds(j * 16, 16)] = dst_v[pl.ds(off + j * 16, 16)]
        pltpu.sync_copy(x_hbm.at[idx_v], rows_v)

        def _scale(g, _c):
            wv = w_v[pl.ds(off + g * 16, 16)]
            for l in range(16):
                sv = jnp.full((16,), wv[l], jnp.float32)
                r = g * 16 + l
                for j in range(D // 16):
                    sl = pl.ds(j * 16, 16)
                    rows_v[r, sl] = rows_v[r, sl] * sv
            return _c
        lax.fori_loop(0, CHUNK // 16, _scale, 0)

        for j in range(CHUNK // 16):
            idx_v[pl.ds(j * 16, 16)] = src_v[pl.ds(off + j * 16, 16)]
        pltpu.sync_copy(rows_v, out_acc.at[idx_v], add=True)
        return _
    lax.fori_loop(0, NCHUNK, _row_chunk, 0)

    plsc.subcore_barrier()

    for k in range(ROWS_PER_TILE // WB):
        r0 = s * ROWS_PER_TILE + k * WB
        pltpu.sync_copy(out_acc.at[pl.ds(r0, WB)], rows_v)
        pltpu.sync_copy(rows_v, p_hbm.at[c, pl.ds(r0, WB)])

    @pl.when(s < DEGW)
    def _():
        pltpu.sync_copy(deg_acc.at[pl.ds(s * 1000, 1000)],
                        w_v.at[pl.ds(0, 1000)])
        pltpu.sync_copy(w_v.at[pl.ds(0, 1000)],
                        deg_hbm.at[pl.ds(c * N + s * 1000, 1000)])


@jax.jit
def _sc_scatter(x, src, dst, w):
    mesh = plsc.VectorSubcoreMesh(core_axis_name="c", subcore_axis_name="s")
    return pl.kernel(
        _sc_body,
        out_type=(
            jax.ShapeDtypeStruct((NC, NPAD, D), jnp.float32),
            jax.ShapeDtypeStruct((NC * N,), jnp.float32),
        ),
        mesh=mesh,
        scratch_types=[
            pltpu.VMEM_SHARED((NPAD, D), jnp.float32),
            pltpu.VMEM_SHARED((N,), jnp.float32),
            pltpu.VMEM((EPW,), jnp.int32),
            pltpu.VMEM((EPW,), jnp.int32),
            pltpu.VMEM((EPW,), jnp.float32),
            pltpu.VMEM((CHUNK, D), jnp.float32),
            pltpu.VMEM((CHUNK,), jnp.int32),
        ],
    )(x, src, dst, w)


def _combine_body(p_ref, deg_ref, x_ref, o_ref):
    d = deg_ref[0] + deg_ref[1] + 1.0
    inv = 1.0 / jnp.maximum(d, 1.0)
    o_ref[...] = (p_ref[0] + p_ref[1] + x_ref[...]) * inv


@jax.jit
def _combine(p, deg, x):
    R = 1000
    deg3 = deg.reshape(NC, N, 1)
    return pl.pallas_call(
        _combine_body,
        grid=(N // R,),
        in_specs=[
            pl.BlockSpec((NC, R, D), lambda i: (0, i, 0)),
            pl.BlockSpec((NC, R, 1), lambda i: (0, i, 0)),
            pl.BlockSpec((R, D), lambda i: (i, 0)),
        ],
        out_specs=pl.BlockSpec((R, D), lambda i: (i, 0)),
        out_shape=jax.ShapeDtypeStruct((N, D), jnp.float32),
    )(p, deg3, x)


def kernel(x, edge_index, edge_weight):
    src = edge_index[0].astype(jnp.int32)
    dst = edge_index[1].astype(jnp.int32)
    w = edge_weight.astype(jnp.float32)
    p, deg = _sc_scatter(x, src, dst, w)
    return _combine(p, deg, x)


# speedup vs baseline: 23.8817x; 1.4568x over previous
import functools

import jax
import jax.numpy as jnp
from jax import lax
from jax.experimental import pallas as pl
from jax.experimental.pallas import tpu as pltpu
from jax.experimental.pallas import tpu_sc as plsc

N = 10000
NPAD = 10240
E = 320000
D = 128
NC = 2
NS = 16
NW = NC * NS
EPW = E // NW
CHUNK = 64
NCHUNK = EPW // CHUNK
TAIL = EPW - NCHUNK * CHUNK
ROWS_PER_TILE = NPAD // NS
WB = CHUNK
DEGW = 10


def _sc_body(x_hbm, src_hbm, dst_hbm, w_hbm, p_hbm, deg_hbm,
             out_acc, deg_acc, src_v, dst_v, w_v, buf0, buf1,
             idxg0, idxg1, idxs, idxt, semg0, semg1):
    c = lax.axis_index("c")
    s = lax.axis_index("s")
    wid = s * NC + c
    base = wid * EPW

    zeros16 = jnp.zeros((16,), jnp.float32)

    def _zero_rows(r, _):
        for j in range(D // 16):
            buf0[r, pl.ds(j * 16, 16)] = zeros16
        return _
    lax.fori_loop(0, WB, _zero_rows, 0)
    for g in range(1024 // 16):
        w_v[pl.ds(g * 16, 16)] = zeros16

    for k in range(ROWS_PER_TILE // WB):
        r0 = s * ROWS_PER_TILE + k * WB
        pltpu.sync_copy(buf0, out_acc.at[pl.ds(r0, WB)])

    @pl.when(s < DEGW)
    def _():
        pltpu.sync_copy(w_v.at[pl.ds(0, 1000)],
                        deg_acc.at[pl.ds(s * 1000, 1000)])

    plsc.subcore_barrier()

    pltpu.sync_copy(src_hbm.at[pl.ds(base, EPW)], src_v)
    pltpu.sync_copy(dst_hbm.at[pl.ds(base, EPW)], dst_v)
    pltpu.sync_copy(w_hbm.at[pl.ds(base, EPW)], w_v)

    def _deg_chunk(ci, _):
        off = ci * CHUNK
        for j in range(CHUNK // 16):
            idxs[pl.ds(j * 16, 16)] = src_v[pl.ds(off + j * 16, 16)]
        pltpu.sync_copy(w_v.at[pl.ds(off, CHUNK)],
                        deg_acc.at[idxs], add=True)
        return _
    lax.fori_loop(0, NCHUNK, _deg_chunk, 0)
    idxt[pl.ds(0, 16)] = src_v[pl.ds(NCHUNK * CHUNK, 16)]
    pltpu.sync_copy(w_v.at[pl.ds(NCHUNK * CHUNK, TAIL)],
                    deg_acc.at[idxt], add=True)

    def _stage(dstref, srcref, off, n):
        for j in range(n // 16):
            dstref[pl.ds(j * 16, 16)] = srcref[pl.ds(off + j * 16, 16)]

    def _scale_buf(buf, off):
        def _scale(g, _c):
            wv = w_v[pl.ds(off + g * 16, 16)]
            for l in range(16):
                sv = jnp.full((16,), wv[l], jnp.float32)
                r = g * 16 + l
                for j in range(D // 16):
                    sl = pl.ds(j * 16, 16)
                    buf[r, sl] = buf[r, sl] * sv
            return _c
        lax.fori_loop(0, CHUNK // 16, _scale, 0)

    def _consume(buf, off):
        _scale_buf(buf, off)
        _stage(idxs, src_v, off, CHUNK)
        pltpu.sync_copy(buf, out_acc.at[idxs], add=True)

    _stage(idxg0, dst_v, 0, CHUNK)
    pltpu.async_copy(x_hbm.at[idxg0], buf0, semg0)

    def _pair(i, carry):
        a = 2 * i * CHUNK
        b = a + CHUNK
        _stage(idxg1, dst_v, b, CHUNK)
        pltpu.async_copy(x_hbm.at[idxg1], buf1, semg1)
        pltpu.make_async_copy(x_hbm.at[idxg0], buf0, semg0).wait()
        _consume(buf0, a)

        @pl.when(i < NCHUNK // 2 - 1)
        def _():
            _stage(idxg0, dst_v, b + CHUNK, CHUNK)
            pltpu.async_copy(x_hbm.at[idxg0], buf0, semg0)
        pltpu.make_async_copy(x_hbm.at[idxg1], buf1, semg1).wait()
        _consume(buf1, b)
        return carry
    lax.fori_loop(0, NCHUNK // 2, _pair, 0)

    toff = NCHUNK * CHUNK
    idxt[pl.ds(0, 16)] = dst_v[pl.ds(toff, 16)]
    pltpu.sync_copy(x_hbm.at[idxt], buf0.at[pl.ds(0, TAIL)])
    wv = w_v[pl.ds(toff, 16)]
    for l in range(16):
        sv = jnp.full((16,), wv[l], jnp.float32)
        for j in range(D // 16):
            sl = pl.ds(j * 16, 16)
            buf0[l, sl] = buf0[l, sl] * sv
    idxt[pl.ds(0, 16)] = src_v[pl.ds(toff, 16)]
    pltpu.sync_copy(buf0.at[pl.ds(0, TAIL)], out_acc.at[idxt], add=True)

    plsc.subcore_barrier()

    for k in range(ROWS_PER_TILE // WB):
        r0 = s * ROWS_PER_TILE + k * WB
        pltpu.sync_copy(out_acc.at[pl.ds(r0, WB)], buf0)
        pltpu.sync_copy(buf0, p_hbm.at[c, pl.ds(r0, WB)])

    @pl.when(s < DEGW)
    def _():
        pltpu.sync_copy(deg_acc.at[pl.ds(s * 1000, 1000)],
                        w_v.at[pl.ds(0, 1000)])
        pltpu.sync_copy(w_v.at[pl.ds(0, 1000)],
                        deg_hbm.at[pl.ds(c * N + s * 1000, 1000)])


@jax.jit
def _sc_scatter(x, src, dst, w):
    mesh = plsc.VectorSubcoreMesh(core_axis_name="c", subcore_axis_name="s")
    return pl.kernel(
        _sc_body,
        out_type=(
            jax.ShapeDtypeStruct((NC, NPAD, D), jnp.float32),
            jax.ShapeDtypeStruct((NC * N,), jnp.float32),
        ),
        mesh=mesh,
        scratch_types=[
            pltpu.VMEM_SHARED((NPAD, D), jnp.float32),
            pltpu.VMEM_SHARED((N,), jnp.float32),
            pltpu.VMEM((EPW,), jnp.int32),
            pltpu.VMEM((EPW,), jnp.int32),
            pltpu.VMEM((EPW,), jnp.float32),
            pltpu.VMEM((CHUNK, D), jnp.float32),
            pltpu.VMEM((CHUNK, D), jnp.float32),
            pltpu.VMEM((CHUNK,), jnp.int32),
            pltpu.VMEM((CHUNK,), jnp.int32),
            pltpu.VMEM((CHUNK,), jnp.int32),
            pltpu.VMEM((16,), jnp.int32),
            pltpu.SemaphoreType.DMA,
            pltpu.SemaphoreType.DMA,
        ],
    )(x, src, dst, w)


def _combine_body(p_ref, deg_ref, x_ref, o_ref):
    d = deg_ref[0] + deg_ref[1] + 1.0
    inv = 1.0 / jnp.maximum(d, 1.0)
    o_ref[...] = (p_ref[0] + p_ref[1] + x_ref[...]) * inv


@jax.jit
def _combine(p, deg, x):
    R = 1000
    deg3 = deg.reshape(NC, N, 1)
    return pl.pallas_call(
        _combine_body,
        grid=(N // R,),
        in_specs=[
            pl.BlockSpec((NC, R, D), lambda i: (0, i, 0)),
            pl.BlockSpec((NC, R, 1), lambda i: (0, i, 0)),
            pl.BlockSpec((R, D), lambda i: (i, 0)),
        ],
        out_specs=pl.BlockSpec((R, D), lambda i: (i, 0)),
        out_shape=jax.ShapeDtypeStruct((N, D), jnp.float32),
    )(p, deg3, x)


def kernel(x, edge_index, edge_weight):
    src = edge_index[0].astype(jnp.int32)
    dst = edge_index[1].astype(jnp.int32)
    w = edge_weight.astype(jnp.float32)
    p, deg = _sc_scatter(x, src, dst, w)
    return _combine(p, deg, x)


# speedup vs baseline: 25.2891x; 1.0589x over previous
import functools

import jax
import jax.numpy as jnp
from jax import lax
from jax.experimental import pallas as pl
from jax.experimental.pallas import tpu as pltpu
from jax.experimental.pallas import tpu_sc as plsc

N = 10000
NPAD = 10240
E = 320000
D = 128
NC = 2
NS = 16
NW = NC * NS
EPW = E // NW
DCH = 64
NDCH = EPW // DCH
GCH = 32
NBUF = 4
NGCH = (EPW // GCH // NBUF) * NBUF
TAIL = EPW - NGCH * GCH
ROWS_PER_TILE = NPAD // NS
WB = GCH
DEGW = 10


def _sc_body(x_hbm, src_hbm, dst_hbm, w_hbm, p_hbm, deg_hbm,
             out_acc, deg_acc, src_v, dst_v, w_v,
             bufs, idxgs, idxs, idxss, idxt, sems):
    c = lax.axis_index("c")
    s = lax.axis_index("s")
    wid = s * NC + c
    base = wid * EPW

    zeros16 = jnp.zeros((16,), jnp.float32)

    def _zero_rows(r, _):
        for j in range(D // 16):
            bufs[0][r, pl.ds(j * 16, 16)] = zeros16
        return _
    lax.fori_loop(0, WB, _zero_rows, 0)
    for g in range(1024 // 16):
        w_v[pl.ds(g * 16, 16)] = zeros16

    for k in range(ROWS_PER_TILE // WB):
        r0 = s * ROWS_PER_TILE + k * WB
        pltpu.sync_copy(bufs[0], out_acc.at[pl.ds(r0, WB)])

    @pl.when(s < DEGW)
    def _():
        pltpu.sync_copy(w_v.at[pl.ds(0, 1000)],
                        deg_acc.at[pl.ds(s * 1000, 1000)])

    plsc.subcore_barrier()

    pltpu.sync_copy(src_hbm.at[pl.ds(base, EPW)], src_v)
    pltpu.sync_copy(dst_hbm.at[pl.ds(base, EPW)], dst_v)
    pltpu.sync_copy(w_hbm.at[pl.ds(base, EPW)], w_v)

    def _deg_chunk(ci, _):
        off = ci * DCH
        for j in range(DCH // 16):
            idxs[pl.ds(j * 16, 16)] = src_v[pl.ds(off + j * 16, 16)]
        pltpu.sync_copy(w_v.at[pl.ds(off, DCH)],
                        deg_acc.at[idxs], add=True)
        return _
    lax.fori_loop(0, NDCH, _deg_chunk, 0)
    idxt[pl.ds(0, 16)] = src_v[pl.ds(NDCH * DCH, 16)]
    pltpu.sync_copy(w_v.at[pl.ds(NDCH * DCH, TAIL)],
                    deg_acc.at[idxt], add=True)

    def _stage(dstref, srcref, off, n):
        for j in range(n // 16):
            dstref[pl.ds(j * 16, 16)] = srcref[pl.ds(off + j * 16, 16)]

    def _scale_buf(buf, off):
        def _scale(g, _c):
            wv = w_v[pl.ds(off + g * 16, 16)]
            for l in range(16):
                sv = jnp.full((16,), wv[l], jnp.float32)
                r = g * 16 + l
                for j in range(D // 16):
                    sl = pl.ds(j * 16, 16)
                    buf[r, sl] = buf[r, sl] * sv
            return _c
        lax.fori_loop(0, GCH // 16, _scale, 0)

    for q in range(NBUF):
        _stage(idxgs[q], dst_v, q * GCH, GCH)
        pltpu.async_copy(x_hbm.at[idxgs[q]], bufs[q], sems[q])

    def _round(i, carry):
        for q in range(NBUF):
            ch = i * NBUF + q
            off = ch * GCH
            pltpu.make_async_copy(x_hbm.at[idxgs[q]], bufs[q],
                                  sems[q]).wait()
            _scale_buf(bufs[q], off)
            _stage(idxss, src_v, off, GCH)
            pltpu.sync_copy(bufs[q], out_acc.at[idxss], add=True)

            @pl.when(ch + NBUF < NGCH)
            def _():
                _stage(idxgs[q], dst_v, off + NBUF * GCH, GCH)
                pltpu.async_copy(x_hbm.at[idxgs[q]], bufs[q], sems[q])
        return carry
    lax.fori_loop(0, NGCH // NBUF, _round, 0)

    toff = NGCH * GCH
    idxt[pl.ds(0, 16)] = dst_v[pl.ds(toff, 16)]
    pltpu.sync_copy(x_hbm.at[idxt], bufs[0].at[pl.ds(0, TAIL)])
    wv = w_v[pl.ds(toff, 16)]
    for l in range(16):
        sv = jnp.full((16,), wv[l], jnp.float32)
        for j in range(D // 16):
            sl = pl.ds(j * 16, 16)
            bufs[0][l, sl] = bufs[0][l, sl] * sv
    idxt[pl.ds(0, 16)] = src_v[pl.ds(toff, 16)]
    pltpu.sync_copy(bufs[0].at[pl.ds(0, TAIL)], out_acc.at[idxt], add=True)

    plsc.subcore_barrier()

    for k in range(ROWS_PER_TILE // WB):
        r0 = s * ROWS_PER_TILE + k * WB
        pltpu.sync_copy(out_acc.at[pl.ds(r0, WB)], bufs[0])
        pltpu.sync_copy(bufs[0], p_hbm.at[c, pl.ds(r0, WB)])

    @pl.when(s < DEGW)
    def _():
        pltpu.sync_copy(deg_acc.at[pl.ds(s * 1000, 1000)],
                        w_v.at[pl.ds(0, 1000)])
        pltpu.sync_copy(w_v.at[pl.ds(0, 1000)],
                        deg_hbm.at[pl.ds(c * N + s * 1000, 1000)])


@jax.jit
def _sc_scatter(x, src, dst, w):
    mesh = plsc.VectorSubcoreMesh(core_axis_name="c", subcore_axis_name="s")
    return pl.kernel(
        _sc_body,
        out_type=(
            jax.ShapeDtypeStruct((NC, NPAD, D), jnp.float32),
            jax.ShapeDtypeStruct((NC * N,), jnp.float32),
        ),
        mesh=mesh,
        scratch_types=[
            pltpu.VMEM_SHARED((NPAD, D), jnp.float32),
            pltpu.VMEM_SHARED((N,), jnp.float32),
            pltpu.VMEM((EPW,), jnp.int32),
            pltpu.VMEM((EPW,), jnp.int32),
            pltpu.VMEM((EPW,), jnp.float32),
            [pltpu.VMEM((GCH, D), jnp.float32) for _ in range(NBUF)],
            [pltpu.VMEM((GCH,), jnp.int32) for _ in range(NBUF)],
            pltpu.VMEM((DCH,), jnp.int32),
            pltpu.VMEM((GCH,), jnp.int32),
            pltpu.VMEM((16,), jnp.int32),
            [pltpu.SemaphoreType.DMA for _ in range(NBUF)],
        ],
    )(x, src, dst, w)


def _combine_body(p_ref, deg_ref, x_ref, o_ref):
    d = deg_ref[0] + deg_ref[1] + 1.0
    inv = 1.0 / jnp.maximum(d, 1.0)
    o_ref[...] = (p_ref[0] + p_ref[1] + x_ref[...]) * inv


@jax.jit
def _combine(p, deg, x):
    R = 1000
    deg3 = deg.reshape(NC, N, 1)
    return pl.pallas_call(
        _combine_body,
        grid=(N // R,),
        in_specs=[
            pl.BlockSpec((NC, R, D), lambda i: (0, i, 0)),
            pl.BlockSpec((NC, R, 1), lambda i: (0, i, 0)),
            pl.BlockSpec((R, D), lambda i: (i, 0)),
        ],
        out_specs=pl.BlockSpec((R, D), lambda i: (i, 0)),
        out_shape=jax.ShapeDtypeStruct((N, D), jnp.float32),
    )(p, deg3, x)


def kernel(x, edge_index, edge_weight):
    src = edge_index[0].astype(jnp.int32)
    dst = edge_index[1].astype(jnp.int32)
    w = edge_weight.astype(jnp.float32)
    p, deg = _sc_scatter(x, src, dst, w)
    return _combine(p, deg, x)
